# Initial kernel scaffold; baseline (speedup 1.0000x reference)
import jax
import jax.numpy as jnp
from jax.experimental import pallas as pl


def kernel(uid_in, iid_in, r_in, v, mu, w, b):
    raise NotImplementedError("write your pallas kernel here")


# speedup vs baseline: 4.3954x; 4.3954x over previous
import functools

import jax
import jax.numpy as jnp
from jax import lax
from jax.experimental import pallas as pl
from jax.experimental.pallas import tpu as pltpu
from jax.experimental.pallas import tpu_sc as plsc

NU = 100000
NI = 100000
D = 32
E = 1600000

NC = 2
NS = 16
L = 16

C_EDGE = 512
NIDX = C_EDGE // 128
NCHUNK = E // C_EDGE

HALF = NI // NC
HREG = 50176
ZSTRIPE = HREG // NS
SROWS = 200
NSCHUNK = HALF // SROWS

_mesh = plsc.VectorSubcoreMesh(
    core_axis_name="c", subcore_axis_name="s", num_cores=NC, num_subcores=NS
)


def _iota16():
    return lax.iota(jnp.int32, L)


def _sigmoid16(x):
    return 1.0 / (1.0 + jnp.exp(-x))


@functools.partial(
    pl.kernel,
    out_type=jax.ShapeDtypeStruct((NI, D), jnp.float32),
    mesh=_mesh,
    compiler_params=pltpu.CompilerParams(needs_layout_passes=False, use_tc_tiling_on_sc=False),
    scratch_types=[
        pltpu.VMEM_SHARED((HREG, D), jnp.float32),
        pltpu.VMEM((C_EDGE, D), jnp.float32),
        pltpu.VMEM((C_EDGE,), jnp.int32),
        pltpu.VMEM((C_EDGE,), jnp.int32),
        pltpu.VMEM((C_EDGE,), jnp.float32),
        pltpu.VMEM((NIDX, 128), jnp.int32),
        pltpu.VMEM((112, D), jnp.float32),
        pltpu.VMEM((SROWS, D), jnp.float32),
        pltpu.VMEM((1, D), jnp.float32),
        pltpu.SemaphoreType.DMA,
    ],
)
def _phase_a(uid1, iid1, r1, v_hbm, mu_hbm, h_out,
             shared, rowsv, uidv, iidv, rv, sidxv, zbuf, sbuf, muv, sem):
    c = lax.axis_index("c")
    s = lax.axis_index("s")
    half_base = c * HALF
    trash = HALF + s

    for i in range(112):
        for hh in range(2):
            zbuf[i, pl.ds(hh * L, L)] = jnp.zeros((L,), jnp.float32)
    z0 = s * ZSTRIPE
    for i in range(ZSTRIPE // 112):
        pltpu.sync_copy(zbuf, shared.at[pl.ds(z0 + i * 112, 112)])
    plsc.subcore_barrier()

    def chunk_body(k, _):
        cid = s + NS * k

        @pl.when(cid < NCHUNK)
        def _():
            base = pl.multiple_of(cid * C_EDGE, 8)
            pltpu.sync_copy(uid1.at[pl.ds(base, C_EDGE)], uidv)
            pltpu.sync_copy(iid1.at[pl.ds(base, C_EDGE)], iidv)
            pltpu.sync_copy(r1.at[pl.ds(base, C_EDGE)], rv)
            descs = [
                pltpu.async_copy(
                    v_hbm.at[uidv.at[pl.ds(j * 128, 128)]],
                    rowsv.at[pl.ds(j * 128, 128)],
                    sem,
                )
                for j in range(NIDX)
            ]
            for t in range(C_EDGE // L):
                ii = iidv[pl.ds(t * L, L)]
                il = ii - half_base
                ok = (il >= 0) & (il < HALF)
                sidxv[t // 8, pl.ds((t % 8) * L, L)] = jnp.where(ok, il, trash)
            for d_ in descs:
                d_.wait()
            def scale_body(g, _):
                re = rv[pl.ds(g * L, L)]
                idx_e = g * L + _iota16()
                for d in range(D):
                    fd = jnp.full((L,), d, jnp.int32)
                    col = plsc.load_gather(rowsv, [idx_e, fd])
                    plsc.store_scatter(rowsv, [idx_e, fd], col * re)
                return 0

            lax.fori_loop(0, C_EDGE // L, scale_body, 0)
            for j in range(NIDX):
                pltpu.sync_copy(
                    rowsv.at[pl.ds(j * 128, 128)],
                    shared.at[sidxv.at[j]],
                    add=True,
                )
        return 0

    lax.fori_loop(0, (NCHUNK - 1) // NS + 1, chunk_body, 0)
    plsc.subcore_barrier()

    pltpu.sync_copy(mu_hbm, muv)

    def sig_body(kk, _):
        cid = s + NS * kk

        @pl.when(cid < NSCHUNK)
        def _():
            row0 = pl.multiple_of(cid * SROWS, 8)
            pltpu.sync_copy(shared.at[pl.ds(row0, SROWS)], sbuf)

            def sig_row(rr, _):
                for hh in range(2):
                    x = sbuf[rr, pl.ds(hh * L, L)] + muv[0, pl.ds(hh * L, L)]
                    sbuf[rr, pl.ds(hh * L, L)] = _sigmoid16(x)
                return 0

            lax.fori_loop(0, SROWS, sig_row, 0)
            pltpu.sync_copy(sbuf, h_out.at[pl.ds(half_base + row0, SROWS)])
        return 0

    lax.fori_loop(0, (NSCHUNK - 1) // NS + 1, sig_body, 0)


@functools.partial(
    pl.kernel,
    out_type=jax.ShapeDtypeStruct((E,), jnp.float32),
    mesh=_mesh,
    compiler_params=pltpu.CompilerParams(needs_layout_passes=False, use_tc_tiling_on_sc=False),
    scratch_types=[
        pltpu.VMEM((C_EDGE, D), jnp.float32),
        pltpu.VMEM((C_EDGE, D), jnp.float32),
        pltpu.VMEM((C_EDGE,), jnp.int32),
        pltpu.VMEM((C_EDGE,), jnp.int32),
        pltpu.VMEM((C_EDGE,), jnp.float32),
        pltpu.VMEM((C_EDGE,), jnp.float32),
        pltpu.SemaphoreType.DMA,
        pltpu.SemaphoreType.DMA,
        pltpu.SemaphoreType.DMA,
    ],
)
def _phase_b(uid1, iid1, h_hbm, w_hbm, b_hbm, out1,
             hrows, wrows, uidv, iidv, bv, outv, semh, semw, semb):
    c = lax.axis_index("c")
    s = lax.axis_index("s")
    wid = s * NC + c
    nw = NC * NS

    def chunk_body(k, _):
        cid = wid + nw * k

        @pl.when(cid < NCHUNK)
        def _():
            base = pl.multiple_of(cid * C_EDGE, 8)
            pltpu.sync_copy(uid1.at[pl.ds(base, C_EDGE)], uidv)
            pltpu.sync_copy(iid1.at[pl.ds(base, C_EDGE)], iidv)
            descs = []
            for j in range(NIDX):
                iidx = iidv.at[pl.ds(j * 128, 128)]
                uidx = uidv.at[pl.ds(j * 128, 128)]
                descs.append(pltpu.async_copy(
                    h_hbm.at[iidx], hrows.at[pl.ds(j * 128, 128)], semh))
                descs.append(pltpu.async_copy(
                    w_hbm.at[uidx], wrows.at[pl.ds(j * 128, 128)], semw))
                descs.append(pltpu.async_copy(
                    b_hbm.at[uidx], bv.at[pl.ds(j * 128, 128)], semb))
            for d_ in descs:
                d_.wait()
            def dot_body(g, _):
                idx_e = g * L + _iota16()
                acc = bv[pl.ds(g * L, L)]
                for d in range(D):
                    fd = jnp.full((L,), d, jnp.int32)
                    hc = plsc.load_gather(hrows, [idx_e, fd])
                    wc = plsc.load_gather(wrows, [idx_e, fd])
                    acc = acc + hc * wc
                outv[pl.ds(g * L, L)] = acc
                return 0

            lax.fori_loop(0, C_EDGE // L, dot_body, 0)
            pltpu.sync_copy(outv, out1.at[pl.ds(base, C_EDGE)])
        return 0

    lax.fori_loop(0, (NCHUNK - 1) // nw + 1, chunk_body, 0)


def kernel(uid_in, iid_in, r_in, v, mu, w, b):
    uid1 = uid_in.astype(jnp.int32)
    iid1 = iid_in.astype(jnp.int32)
    h = _phase_a(uid1, iid1, r_in, v, mu)
    return _phase_b(uid1, iid1, h, w, b)


# speedup vs baseline: 6.0382x; 1.3737x over previous
import functools

import jax
import jax.numpy as jnp
from jax import lax
from jax.experimental import pallas as pl
from jax.experimental.pallas import tpu as pltpu
from jax.experimental.pallas import tpu_sc as plsc

NU = 100000
NI = 100000
D = 32
E = 1600000

NC = 2
NS = 16
L = 16

C_EDGE = 512
NIDX = C_EDGE // 128
NCHUNK = E // C_EDGE

HREGB = 100000
ZSTRIPE = HREGB // NS
ZROWS = 250
CBLK = 4000

_mesh = plsc.VectorSubcoreMesh(
    core_axis_name="c", subcore_axis_name="s", num_cores=NC, num_subcores=NS
)
_params = pltpu.CompilerParams(
    needs_layout_passes=False, use_tc_tiling_on_sc=False
)


def _iota16():
    return lax.iota(jnp.int32, L)


@functools.partial(
    pl.kernel,
    out_type=jax.ShapeDtypeStruct((NC * HREGB, D), jnp.bfloat16),
    mesh=_mesh,
    compiler_params=_params,
    scratch_types=[
        pltpu.VMEM_SHARED((HREGB, D), jnp.bfloat16),
        pltpu.VMEM((C_EDGE, D), jnp.float32),
        pltpu.VMEM((C_EDGE, D), jnp.bfloat16),
        pltpu.VMEM((C_EDGE,), jnp.int32),
        pltpu.VMEM((NIDX, 128), jnp.int32),
        pltpu.VMEM((C_EDGE,), jnp.float32),
        pltpu.VMEM((ZROWS, D), jnp.bfloat16),
        pltpu.SemaphoreType.DMA,
    ],
)
def _phase_a(uid1, iid1, r1, v_hbm, part_out,
             shared, rowsv, brows, uidv, sidxv, rv, zbuf, sem):
    c = lax.axis_index("c")
    s = lax.axis_index("s")

    def zfill(i, _):
        zbuf[i, :] = jnp.zeros((2 * L,), jnp.bfloat16)
        return 0

    lax.fori_loop(0, ZROWS, zfill, 0)
    z0 = s * ZSTRIPE
    for i in range(ZSTRIPE // ZROWS):
        pltpu.sync_copy(zbuf, shared.at[pl.ds(z0 + i * ZROWS, ZROWS)])
    plsc.subcore_barrier()

    def chunk_body(k, _):
        cid = c + 2 * s + 32 * k

        @pl.when(cid < NCHUNK)
        def _():
            base = pl.multiple_of(cid * C_EDGE, 8)
            pltpu.sync_copy(uid1.at[pl.ds(base, C_EDGE)], uidv)
            pltpu.sync_copy(r1.at[pl.ds(base, C_EDGE)], rv)
            for j in range(NIDX):
                pltpu.sync_copy(
                    iid1.at[pl.ds(base + j * 128, 128)], sidxv.at[j]
                )
            descs = [
                pltpu.async_copy(
                    v_hbm.at[uidv.at[pl.ds(j * 128, 128)]],
                    rowsv.at[pl.ds(j * 128, 128)],
                    sem,
                )
                for j in range(NIDX)
            ]
            for d_ in descs:
                d_.wait()

            def scale_body(g, _):
                re = rv[pl.ds(g * L, L)]
                idx_e = g * L + _iota16()
                for d in range(D):
                    fd = jnp.full((L,), d, jnp.int32)
                    col = plsc.load_gather(rowsv, [idx_e, fd])
                    plsc.store_scatter(rowsv, [idx_e, fd], col * re)
                return 0

            lax.fori_loop(0, C_EDGE // L, scale_body, 0)

            def pack_body(e, _):
                se = e + jnp.zeros((L,), jnp.int32)
                a = plsc.load_gather(rowsv, [se, _iota16()])
                b = plsc.load_gather(rowsv, [se, L + _iota16()])
                brows[e, :] = plsc.pack(
                    a, b, format=plsc.PackFormat.INTERLEAVED
                )
                return 0

            lax.fori_loop(0, C_EDGE, pack_body, 0)

            for j in range(NIDX):
                pltpu.sync_copy(
                    brows.at[pl.ds(j * 128, 128)],
                    shared.at[sidxv.at[j]],
                    add=True,
                )
        return 0

    lax.fori_loop(0, (NCHUNK - 1) // (NC * NS) + 1, chunk_body, 0)
    plsc.subcore_barrier()

    pltpu.sync_copy(
        shared.at[pl.ds(z0, ZSTRIPE)],
        part_out.at[pl.ds(c * HREGB + z0, ZSTRIPE)],
    )


@functools.partial(
    pl.pallas_call,
    out_shape=jax.ShapeDtypeStruct((NI, D), jnp.float32),
    grid=(NI // CBLK,),
    in_specs=[
        pl.BlockSpec((CBLK, D), lambda i: (i, 0)),
        pl.BlockSpec((CBLK, D), lambda i: (i + HREGB // CBLK, 0)),
        pl.BlockSpec((1, D), lambda i: (0, 0)),
    ],
    out_specs=pl.BlockSpec((CBLK, D), lambda i: (i, 0)),
)
def _combine(p0, p1, mu, h_out):
    x = p0[...].astype(jnp.float32) + p1[...].astype(jnp.float32) + mu[...]
    h_out[...] = 1.0 / (1.0 + jnp.exp(-x))


@functools.partial(
    pl.kernel,
    out_type=jax.ShapeDtypeStruct((E,), jnp.float32),
    mesh=_mesh,
    compiler_params=_params,
    scratch_types=[
        pltpu.VMEM((C_EDGE, D), jnp.float32),
        pltpu.VMEM((C_EDGE, D), jnp.float32),
        pltpu.VMEM((C_EDGE,), jnp.int32),
        pltpu.VMEM((C_EDGE,), jnp.int32),
        pltpu.VMEM((C_EDGE,), jnp.float32),
        pltpu.VMEM((C_EDGE,), jnp.float32),
        pltpu.SemaphoreType.DMA,
        pltpu.SemaphoreType.DMA,
        pltpu.SemaphoreType.DMA,
    ],
)
def _phase_b(uid1, iid1, h_hbm, w_hbm, b_hbm, out1,
             hrows, wrows, uidv, iidv, bv, outv, semh, semw, semb):
    c = lax.axis_index("c")
    s = lax.axis_index("s")
    wid = s * NC + c
    nw = NC * NS

    def chunk_body(k, _):
        cid = wid + nw * k

        @pl.when(cid < NCHUNK)
        def _():
            base = pl.multiple_of(cid * C_EDGE, 8)
            pltpu.sync_copy(uid1.at[pl.ds(base, C_EDGE)], uidv)
            pltpu.sync_copy(iid1.at[pl.ds(base, C_EDGE)], iidv)
            descs = []
            for j in range(NIDX):
                iidx = iidv.at[pl.ds(j * 128, 128)]
                uidx = uidv.at[pl.ds(j * 128, 128)]
                descs.append(pltpu.async_copy(
                    h_hbm.at[iidx], hrows.at[pl.ds(j * 128, 128)], semh))
                descs.append(pltpu.async_copy(
                    w_hbm.at[uidx], wrows.at[pl.ds(j * 128, 128)], semw))
                descs.append(pltpu.async_copy(
                    b_hbm.at[uidx], bv.at[pl.ds(j * 128, 128)], semb))
            for d_ in descs:
                d_.wait()

            def dot_body(g, _):
                idx_e = g * L + _iota16()
                acc = bv[pl.ds(g * L, L)]
                for d in range(D):
                    fd = jnp.full((L,), d, jnp.int32)
                    hc = plsc.load_gather(hrows, [idx_e, fd])
                    wc = plsc.load_gather(wrows, [idx_e, fd])
                    acc = acc + hc * wc
                outv[pl.ds(g * L, L)] = acc
                return 0

            lax.fori_loop(0, C_EDGE // L, dot_body, 0)
            pltpu.sync_copy(outv, out1.at[pl.ds(base, C_EDGE)])
        return 0

    lax.fori_loop(0, (NCHUNK - 1) // nw + 1, chunk_body, 0)


def _interleave_cols(x):
    return jnp.stack([x[:, :L], x[:, L:]], axis=-1).reshape(x.shape[0], D)


def kernel(uid_in, iid_in, r_in, v, mu, w, b):
    uid1 = uid_in.astype(jnp.int32)
    iid1 = iid_in.astype(jnp.int32)
    parts = _phase_a(uid1, iid1, r_in, v)
    h = _combine(parts, parts, _interleave_cols(mu))
    wp = _interleave_cols(w)
    return _phase_b(uid1, iid1, h, wp, b)


# speedup vs baseline: 7.4188x; 1.2286x over previous
import functools

import jax
import jax.numpy as jnp
from jax import lax
from jax.experimental import pallas as pl
from jax.experimental.pallas import tpu as pltpu
from jax.experimental.pallas import tpu_sc as plsc

NU = 100000
NI = 100000
D = 32
E = 1600000

NC = 2
NS = 16
L = 16
NW = NC * NS
EG = E // 128

CA = 256
ANIDX = CA // 128
ANCHUNK = E // CA
ANITER = (ANCHUNK - 1) // NW + 1

CB = 512
BNIDX = CB // 128
BNCHUNK = E // CB
BNITER = (BNCHUNK - 1) // NW + 1

HREGB = 100000
ZSTRIPE = HREGB // NS
ZROWS = 50
CBLK = 4000
WB = 48

_mesh = plsc.VectorSubcoreMesh(
    core_axis_name="c", subcore_axis_name="s", num_cores=NC, num_subcores=NS
)
_params = pltpu.CompilerParams(
    needs_layout_passes=False, use_tc_tiling_on_sc=False
)


def _iota16():
    return lax.iota(jnp.int32, L)


@functools.partial(
    pl.kernel,
    out_type=jax.ShapeDtypeStruct((NC * HREGB, D), jnp.bfloat16),
    mesh=_mesh,
    compiler_params=_params,
    scratch_types=[
        pltpu.VMEM_SHARED((HREGB, D), jnp.bfloat16),
        pltpu.VMEM((CA, D), jnp.float32),
        pltpu.VMEM((CA, D), jnp.float32),
        pltpu.VMEM((CA, D), jnp.bfloat16),
        pltpu.VMEM((CA, D), jnp.bfloat16),
        pltpu.VMEM((3 * CA,), jnp.int32),
        pltpu.VMEM((3 * CA,), jnp.int32),
        pltpu.VMEM((ANIDX, 128), jnp.int32),
        pltpu.VMEM((ANIDX, 128), jnp.int32),
        pltpu.VMEM((ZROWS, D), jnp.bfloat16),
        pltpu.SemaphoreType.DMA,
        pltpu.SemaphoreType.DMA,
        pltpu.SemaphoreType.DMA,
        pltpu.SemaphoreType.DMA,
    ],
)
def _phase_a(pk3, v_hbm, part_out,
             shared, rows0, rows1, br0, br1, pk0, pkb1,
             si0, si1, zbuf, sg0, sg1, ss0, ss1):
    c = lax.axis_index("c")
    s = lax.axis_index("s")
    rowsv = (rows0, rows1)
    brows = (br0, br1)
    pkv = (pk0, pkb1)
    sidxv = (si0, si1)
    semg = (sg0, sg1)
    sems = (ss0, ss1)

    def cid_of(k):
        return c + 2 * s + NW * k

    def front(k, b):
        cid = cid_of(k)

        @pl.when(cid < ANCHUNK)
        def _():
            b768 = pl.multiple_of(cid * 3 * CA, 8)
            pltpu.sync_copy(pk3.at[pl.ds(b768, 3 * CA)], pkv[b])
            for j in range(ANIDX):
                pltpu.async_copy(
                    v_hbm.at[pkv[b].at[pl.ds(j * 384, 128)]],
                    rowsv[b].at[pl.ds(j * 128, 128)],
                    semg[b],
                )

    def wait_gathers(b):
        for j in range(ANIDX):
            pltpu.make_async_copy(
                v_hbm.at[pkv[b].at[pl.ds(j * 384, 128)]],
                rowsv[b].at[pl.ds(j * 128, 128)],
                semg[b],
            ).wait()

    def wait_scatters(b):
        for j in range(ANIDX):
            pltpu.make_async_copy(
                brows[b].at[pl.ds(j * 128, 128)],
                shared.at[sidxv[b].at[j]],
                sems[b],
            ).wait()

    def zfill(i, _):
        zbuf[i, :] = jnp.zeros((2 * L,), jnp.bfloat16)
        return 0

    lax.fori_loop(0, ZROWS, zfill, 0)
    z0 = s * ZSTRIPE

    def zcopy(i, _):
        pltpu.sync_copy(zbuf, shared.at[pl.ds(z0 + i * ZROWS, ZROWS)])
        return 0

    lax.fori_loop(0, ZSTRIPE // ZROWS, zcopy, 0)
    plsc.subcore_barrier()

    front(0, 0)

    def round_body(m, _):
        for b in (0, 1):
            k = 2 * m + b
            cid = cid_of(k)

            @pl.when(cid < ANCHUNK)
            def _():
                wait_gathers(b)
                front(k + 1, 1 - b)

                def scale_body(g, _):
                    roff = (g >> 3) * 384 + 256 + (g & 7) * L
                    re = plsc.bitcast(pkv[b][pl.ds(roff, L)], jnp.float32)
                    idx_e = g * L + _iota16()
                    for d in range(D):
                        fd = jnp.full((L,), d, jnp.int32)
                        col = plsc.load_gather(rowsv[b], [idx_e, fd])
                        plsc.store_scatter(rowsv[b], [idx_e, fd], col * re)
                    return 0

                lax.fori_loop(0, CA // L, scale_body, 0)

                @pl.when(k >= 2)
                def _():
                    wait_scatters(b)

                for t in range(CA // L):
                    j, tt = t // 8, t % 8
                    sidxv[b][j, pl.ds(tt * L, L)] = (
                        pkv[b][pl.ds(j * 384 + 128 + tt * L, L)]
                    )

                def pack_body(eo, _):
                    for ei in range(8):
                        e = eo * 8 + ei
                        se = e + jnp.zeros((L,), jnp.int32)
                        a = plsc.load_gather(rowsv[b], [se, _iota16()])
                        bb = plsc.load_gather(rowsv[b], [se, L + _iota16()])
                        brows[b][e, :] = plsc.pack(
                            a, bb, format=plsc.PackFormat.INTERLEAVED
                        )
                    return 0

                lax.fori_loop(0, CA // 8, pack_body, 0)

                for j in range(ANIDX):
                    pltpu.async_copy(
                        brows[b].at[pl.ds(j * 128, 128)],
                        shared.at[sidxv[b].at[j]],
                        sems[b],
                        add=True,
                    )
        return 0

    lax.fori_loop(0, ANITER // 2, round_body, 0)
    for b in (0, 1):
        wait_scatters(b)

    plsc.subcore_barrier()

    pltpu.sync_copy(
        shared.at[pl.ds(z0, ZSTRIPE)],
        part_out.at[pl.ds(c * HREGB + z0, ZSTRIPE)],
    )


@functools.partial(
    pl.pallas_call,
    out_shape=jax.ShapeDtypeStruct((NI, D), jnp.float32),
    grid=(NI // CBLK,),
    in_specs=[
        pl.BlockSpec((CBLK, D), lambda i: (i, 0)),
        pl.BlockSpec((CBLK, D), lambda i: (i + HREGB // CBLK, 0)),
        pl.BlockSpec((1, D), lambda i: (0, 0)),
    ],
    out_specs=pl.BlockSpec((CBLK, D), lambda i: (i, 0)),
)
def _combine(p0, p1, mu, h_out):
    x = p0[...].astype(jnp.float32) + p1[...].astype(jnp.float32) + mu[...]
    h_out[...] = 1.0 / (1.0 + jnp.exp(-x))


@functools.partial(
    pl.kernel,
    out_type=jax.ShapeDtypeStruct((E,), jnp.float32),
    mesh=_mesh,
    compiler_params=_params,
    scratch_types=[
        pltpu.VMEM((CB, D), jnp.float32),
        pltpu.VMEM((CB, D), jnp.float32),
        pltpu.VMEM((CB, WB), jnp.float32),
        pltpu.VMEM((CB, WB), jnp.float32),
        pltpu.VMEM((3 * CB,), jnp.int32),
        pltpu.VMEM((3 * CB,), jnp.int32),
        pltpu.VMEM((CB,), jnp.float32),
        pltpu.SemaphoreType.DMA,
        pltpu.SemaphoreType.DMA,
    ],
)
def _phase_b(pk3, h_hbm, wb_hbm, out1,
             h0, h1, w0, w1, pk0, pkb1, outv, sg0, sg1):
    c = lax.axis_index("c")
    s = lax.axis_index("s")
    wid = s * NC + c
    hrows = (h0, h1)
    wrows = (w0, w1)
    pkv = (pk0, pkb1)
    semg = (sg0, sg1)

    def cid_of(k):
        return wid + NW * k

    def front(k, b):
        cid = cid_of(k)

        @pl.when(cid < BNCHUNK)
        def _():
            bo = pl.multiple_of(cid * 3 * CB, 8)
            pltpu.sync_copy(pk3.at[pl.ds(bo, 3 * CB)], pkv[b])
            for j in range(BNIDX):
                pltpu.async_copy(
                    h_hbm.at[pkv[b].at[pl.ds(j * 384 + 128, 128)]],
                    hrows[b].at[pl.ds(j * 128, 128)],
                    semg[b],
                )
                pltpu.async_copy(
                    wb_hbm.at[pkv[b].at[pl.ds(j * 384, 128)]],
                    wrows[b].at[pl.ds(j * 128, 128)],
                    semg[b],
                )

    def wait_gathers(b):
        pltpu.make_async_copy(
            h_hbm.at[pl.ds(0, CB)], hrows[b], semg[b]
        ).wait()
        pltpu.make_async_copy(
            wb_hbm.at[pl.ds(0, CB)], wrows[b], semg[b]
        ).wait()

    front(0, 0)

    def round_body(m, _):
        for b in (0, 1):
            k = 2 * m + b
            cid = cid_of(k)

            @pl.when(cid < BNCHUNK)
            def _():
                wait_gathers(b)
                front(k + 1, 1 - b)

                def dot_body(g, _):
                    idx_e = g * L + _iota16()
                    fb = jnp.full((L,), D, jnp.int32)
                    acc = plsc.load_gather(wrows[b], [idx_e, fb])
                    for d in range(D):
                        fd = jnp.full((L,), d, jnp.int32)
                        hc = plsc.load_gather(hrows[b], [idx_e, fd])
                        wc = plsc.load_gather(wrows[b], [idx_e, fd])
                        acc = acc + hc * wc
                    outv[pl.ds(g * L, L)] = acc
                    return 0

                lax.fori_loop(0, CB // L, dot_body, 0)
                base = pl.multiple_of(cid * CB, 8)
                pltpu.sync_copy(outv, out1.at[pl.ds(base, CB)])
        return 0

    lax.fori_loop(0, BNITER // 2, round_body, 0)


def _interleave_cols(x):
    return jnp.stack([x[:, :L], x[:, L:]], axis=-1).reshape(x.shape[0], D)


def kernel(uid_in, iid_in, r_in, v, mu, w, b):
    uid1 = uid_in.astype(jnp.int32)
    iid1 = iid_in.astype(jnp.int32)
    pk3 = jnp.stack(
        [
            uid1.reshape(EG, 128),
            iid1.reshape(EG, 128),
            lax.bitcast_convert_type(r_in, jnp.int32).reshape(EG, 128),
        ],
        axis=1,
    ).reshape(3 * E)
    parts = _phase_a(pk3, v)
    h = _combine(parts, parts, _interleave_cols(mu))
    wb = jnp.concatenate(
        [
            _interleave_cols(w),
            b[:, None],
            jnp.zeros((NU, WB - D - 1), jnp.float32),
        ],
        axis=1,
    )
    return _phase_b(pk3, h, wb)


# speedup vs baseline: 14.5786x; 1.9651x over previous
import functools

import jax
import jax.numpy as jnp
from jax import lax
from jax.experimental import pallas as pl
from jax.experimental.pallas import tpu as pltpu
from jax.experimental.pallas import tpu_sc as plsc

NU = 100000
NI = 100000
D = 32
E = 1600000

NC = 2
NS = 16
L = 16
NW = NC * NS
EG = E // 128

CA = 256
ANIDX = CA // 128
ANCHUNK = E // CA
ANITER = (ANCHUNK - 1) // NW + 1

CB = 512
BNIDX = CB // 128
BNCHUNK = E // CB
BNITER = (BNCHUNK - 1) // NW + 1

HREGB = 100000
ZSTRIPE = HREGB // NS
ZROWS = 50
CBLK = 4000
WB = 48

_mesh = plsc.VectorSubcoreMesh(
    core_axis_name="c", subcore_axis_name="s", num_cores=NC, num_subcores=NS
)
_params = pltpu.CompilerParams(
    needs_layout_passes=False, use_tc_tiling_on_sc=False
)


def _iota16():
    return lax.iota(jnp.int32, L)


@functools.partial(
    pl.kernel,
    out_type=jax.ShapeDtypeStruct((NC * HREGB, D), jnp.bfloat16),
    mesh=_mesh,
    compiler_params=_params,
    scratch_types=[
        pltpu.VMEM_SHARED((HREGB, D), jnp.bfloat16),
        pltpu.VMEM((CA, D), jnp.bfloat16),
        pltpu.VMEM((CA, D), jnp.bfloat16),
        pltpu.VMEM((CA, D), jnp.bfloat16),
        pltpu.VMEM((CA, D), jnp.bfloat16),
        pltpu.VMEM((3 * CA,), jnp.int32),
        pltpu.VMEM((3 * CA,), jnp.int32),
        pltpu.VMEM((ANIDX, 128), jnp.int32),
        pltpu.VMEM((ANIDX, 128), jnp.int32),
        pltpu.VMEM((ZROWS, D), jnp.bfloat16),
        pltpu.SemaphoreType.DMA,
        pltpu.SemaphoreType.DMA,
        pltpu.SemaphoreType.DMA,
        pltpu.SemaphoreType.DMA,
    ],
)
def _phase_a(pk3, v_hbm, part_out,
             shared, rows0, rows1, br0, br1, pk0, pkb1,
             si0, si1, zbuf, sg0, sg1, ss0, ss1):
    c = lax.axis_index("c")
    s = lax.axis_index("s")
    rowsv = (rows0, rows1)
    brows = (br0, br1)
    pkv = (pk0, pkb1)
    sidxv = (si0, si1)
    semg = (sg0, sg1)
    sems = (ss0, ss1)

    def cid_of(k):
        return c + 2 * s + NW * k

    def front(k, b):
        cid = cid_of(k)

        @pl.when(cid < ANCHUNK)
        def _():
            b768 = pl.multiple_of(cid * 3 * CA, 8)
            pltpu.sync_copy(pk3.at[pl.ds(b768, 3 * CA)], pkv[b])
            for j in range(ANIDX):
                pltpu.async_copy(
                    v_hbm.at[pkv[b].at[pl.ds(j * 384, 128)]],
                    rowsv[b].at[pl.ds(j * 128, 128)],
                    semg[b],
                )

    def wait_gathers(b):
        for j in range(ANIDX):
            pltpu.make_async_copy(
                v_hbm.at[pkv[b].at[pl.ds(j * 384, 128)]],
                rowsv[b].at[pl.ds(j * 128, 128)],
                semg[b],
            ).wait()

    def wait_scatters(b):
        for j in range(ANIDX):
            pltpu.make_async_copy(
                brows[b].at[pl.ds(j * 128, 128)],
                shared.at[sidxv[b].at[j]],
                sems[b],
            ).wait()

    def zfill(i, _):
        zbuf[i, :] = jnp.zeros((2 * L,), jnp.bfloat16)
        return 0

    lax.fori_loop(0, ZROWS, zfill, 0)
    z0 = s * ZSTRIPE

    def zcopy(i, _):
        pltpu.sync_copy(zbuf, shared.at[pl.ds(z0 + i * ZROWS, ZROWS)])
        return 0

    lax.fori_loop(0, ZSTRIPE // ZROWS, zcopy, 0)
    plsc.subcore_barrier()

    front(0, 0)

    def round_body(m, _):
        for b in (0, 1):
            k = 2 * m + b
            cid = cid_of(k)

            @pl.when(cid < ANCHUNK)
            def _():
                wait_gathers(b)
                front(k + 1, 1 - b)

                @pl.when(k >= 2)
                def _():
                    wait_scatters(b)

                for t in range(CA // L):
                    j, tt = t // 8, t % 8
                    sidxv[b][j, pl.ds(tt * L, L)] = (
                        pkv[b][pl.ds(j * 384 + 128 + tt * L, L)]
                    )

                def fuse_body(eo, _):
                    for ei in range(4):
                        e = eo * 4 + ei
                        roff = (e >> 7) * 384 + 256 + (e & 127)
                        sr = roff + jnp.zeros((L,), jnp.int32)
                        rf = plsc.bitcast(
                            plsc.load_gather(pkv[b], [sr]), jnp.float32
                        )
                        rp = plsc.pack(
                            rf, rf, format=plsc.PackFormat.INTERLEAVED
                        )
                        brows[b][e, :] = rowsv[b][e, :] * rp
                    return 0

                lax.fori_loop(0, CA // 4, fuse_body, 0)

                for j in range(ANIDX):
                    pltpu.async_copy(
                        brows[b].at[pl.ds(j * 128, 128)],
                        shared.at[sidxv[b].at[j]],
                        sems[b],
                        add=True,
                    )
        return 0

    lax.fori_loop(0, ANITER // 2, round_body, 0)
    for b in (0, 1):
        wait_scatters(b)

    plsc.subcore_barrier()

    pltpu.sync_copy(
        shared.at[pl.ds(z0, ZSTRIPE)],
        part_out.at[pl.ds(c * HREGB + z0, ZSTRIPE)],
    )


@functools.partial(
    pl.pallas_call,
    out_shape=jax.ShapeDtypeStruct((NI, D), jnp.float32),
    grid=(NI // CBLK,),
    in_specs=[
        pl.BlockSpec((CBLK, D), lambda i: (i, 0)),
        pl.BlockSpec((CBLK, D), lambda i: (i + HREGB // CBLK, 0)),
        pl.BlockSpec((1, D), lambda i: (0, 0)),
    ],
    out_specs=pl.BlockSpec((CBLK, D), lambda i: (i, 0)),
)
def _combine(p0, p1, mu, h_out):
    x = p0[...].astype(jnp.float32) + p1[...].astype(jnp.float32) + mu[...]
    h_out[...] = 1.0 / (1.0 + jnp.exp(-x))


@functools.partial(
    pl.kernel,
    out_type=jax.ShapeDtypeStruct((E,), jnp.float32),
    mesh=_mesh,
    compiler_params=_params,
    scratch_types=[
        pltpu.VMEM((CB, D), jnp.float32),
        pltpu.VMEM((CB, D), jnp.float32),
        pltpu.VMEM((CB, WB), jnp.float32),
        pltpu.VMEM((CB, WB), jnp.float32),
        pltpu.VMEM((3 * CB,), jnp.int32),
        pltpu.VMEM((3 * CB,), jnp.int32),
        pltpu.VMEM((CB,), jnp.float32),
        pltpu.SemaphoreType.DMA,
        pltpu.SemaphoreType.DMA,
    ],
)
def _phase_b(pk3, h_hbm, wb_hbm, out1,
             h0, h1, w0, w1, pk0, pkb1, outv, sg0, sg1):
    c = lax.axis_index("c")
    s = lax.axis_index("s")
    wid = s * NC + c
    hrows = (h0, h1)
    wrows = (w0, w1)
    pkv = (pk0, pkb1)
    semg = (sg0, sg1)

    def cid_of(k):
        return wid + NW * k

    def front(k, b):
        cid = cid_of(k)

        @pl.when(cid < BNCHUNK)
        def _():
            bo = pl.multiple_of(cid * 3 * CB, 8)
            pltpu.sync_copy(pk3.at[pl.ds(bo, 3 * CB)], pkv[b])
            for j in range(BNIDX):
                pltpu.async_copy(
                    h_hbm.at[pkv[b].at[pl.ds(j * 384 + 128, 128)]],
                    hrows[b].at[pl.ds(j * 128, 128)],
                    semg[b],
                )
                pltpu.async_copy(
                    wb_hbm.at[pkv[b].at[pl.ds(j * 384, 128)]],
                    wrows[b].at[pl.ds(j * 128, 128)],
                    semg[b],
                )

    def wait_gathers(b):
        pltpu.make_async_copy(
            h_hbm.at[pl.ds(0, CB)], hrows[b], semg[b]
        ).wait()
        pltpu.make_async_copy(
            wb_hbm.at[pl.ds(0, CB)], wrows[b], semg[b]
        ).wait()

    front(0, 0)

    def round_body(m, _):
        for b in (0, 1):
            k = 2 * m + b
            cid = cid_of(k)

            @pl.when(cid < BNCHUNK)
            def _():
                wait_gathers(b)
                front(k + 1, 1 - b)

                def dot_body(g, _):
                    idx_e = g * L + _iota16()
                    fb = jnp.full((L,), D, jnp.int32)
                    acc = plsc.load_gather(wrows[b], [idx_e, fb])
                    for d in range(D):
                        fd = jnp.full((L,), d, jnp.int32)
                        hc = plsc.load_gather(hrows[b], [idx_e, fd])
                        wc = plsc.load_gather(wrows[b], [idx_e, fd])
                        acc = acc + hc * wc
                    outv[pl.ds(g * L, L)] = acc
                    return 0

                lax.fori_loop(0, CB // L, dot_body, 0)
                base = pl.multiple_of(cid * CB, 8)
                pltpu.sync_copy(outv, out1.at[pl.ds(base, CB)])
        return 0

    lax.fori_loop(0, BNITER // 2, round_body, 0)


def kernel(uid_in, iid_in, r_in, v, mu, w, b):
    uid1 = uid_in.astype(jnp.int32)
    iid1 = iid_in.astype(jnp.int32)
    pk3 = jnp.stack(
        [
            uid1.reshape(EG, 128),
            iid1.reshape(EG, 128),
            lax.bitcast_convert_type(r_in, jnp.int32).reshape(EG, 128),
        ],
        axis=1,
    ).reshape(3 * E)
    parts = _phase_a(pk3, v.astype(jnp.bfloat16))
    h = _combine(parts, parts, mu)
    wb = jnp.concatenate(
        [
            w,
            b[:, None],
            jnp.zeros((NU, WB - D - 1), jnp.float32),
        ],
        axis=1,
    )
    return _phase_b(pk3, h, wb)


# speedup vs baseline: 14.6357x; 1.0039x over previous
import functools

import jax
import jax.numpy as jnp
from jax import lax
from jax.experimental import pallas as pl
from jax.experimental.pallas import tpu as pltpu
from jax.experimental.pallas import tpu_sc as plsc

NU = 100000
NI = 100000
D = 32
E = 1600000

NC = 2
NS = 16
L = 16
NW = NC * NS
EG = E // 128

CA = 256
ANIDX = CA // 128
ANCHUNK = E // CA
ANITER = (ANCHUNK - 1) // NW + 1

CB = 512
BNIDX = CB // 128
BNCHUNK = E // CB
BNITER = (BNCHUNK - 1) // NW + 1

HREGB = 100000
ZSTRIPE = HREGB // NS
ZROWS = 50
CBLK = 4000
WB = 48

_mesh = plsc.VectorSubcoreMesh(
    core_axis_name="c", subcore_axis_name="s", num_cores=NC, num_subcores=NS
)
_params = pltpu.CompilerParams(
    needs_layout_passes=False, use_tc_tiling_on_sc=False
)


def _iota16():
    return lax.iota(jnp.int32, L)


@functools.partial(
    pl.kernel,
    out_type=jax.ShapeDtypeStruct((NC * HREGB, D), jnp.bfloat16),
    mesh=_mesh,
    compiler_params=_params,
    scratch_types=[
        pltpu.VMEM_SHARED((HREGB, D), jnp.bfloat16),
        pltpu.VMEM((CA, D), jnp.bfloat16),
        pltpu.VMEM((CA, D), jnp.bfloat16),
        pltpu.VMEM((CA, D), jnp.bfloat16),
        pltpu.VMEM((CA, D), jnp.bfloat16),
        pltpu.VMEM((3 * CA,), jnp.int32),
        pltpu.VMEM((3 * CA,), jnp.int32),
        pltpu.VMEM((ANIDX, 128), jnp.int32),
        pltpu.VMEM((ANIDX, 128), jnp.int32),
        pltpu.VMEM((ZROWS, D), jnp.bfloat16),
        pltpu.SemaphoreType.DMA,
        pltpu.SemaphoreType.DMA,
        pltpu.SemaphoreType.DMA,
        pltpu.SemaphoreType.DMA,
        pltpu.SemaphoreType.DMA,
        pltpu.SemaphoreType.DMA,
    ],
)
def _phase_a(pk3, v_hbm, part_out,
             shared, rows0, rows1, br0, br1, pk0, pkb1,
             si0, si1, zbuf, sg0, sg1, ss0, ss1, sp0, sp1):
    c = lax.axis_index("c")
    s = lax.axis_index("s")
    rowsv = (rows0, rows1)
    brows = (br0, br1)
    pkv = (pk0, pkb1)
    sidxv = (si0, si1)
    semg = (sg0, sg1)
    sems = (ss0, ss1)
    sempk = (sp0, sp1)

    def cid_of(k):
        return c + 2 * s + NW * k

    def front_pk(k, b):
        cid = cid_of(k)

        @pl.when(cid < ANCHUNK)
        def _():
            b768 = pl.multiple_of(cid * 3 * CA, 8)
            pltpu.async_copy(pk3.at[pl.ds(b768, 3 * CA)], pkv[b], sempk[b])

    def front(k, b):
        cid = cid_of(k)

        @pl.when(cid < ANCHUNK)
        def _():
            b768 = pl.multiple_of(cid * 3 * CA, 8)
            pltpu.make_async_copy(
                pk3.at[pl.ds(b768, 3 * CA)], pkv[b], sempk[b]
            ).wait()
            for j in range(ANIDX):
                pltpu.async_copy(
                    v_hbm.at[pkv[b].at[pl.ds(j * 384, 128)]],
                    rowsv[b].at[pl.ds(j * 128, 128)],
                    semg[b],
                )

    def wait_gathers(b):
        for j in range(ANIDX):
            pltpu.make_async_copy(
                v_hbm.at[pkv[b].at[pl.ds(j * 384, 128)]],
                rowsv[b].at[pl.ds(j * 128, 128)],
                semg[b],
            ).wait()

    def wait_scatters(b):
        for j in range(ANIDX):
            pltpu.make_async_copy(
                brows[b].at[pl.ds(j * 128, 128)],
                shared.at[sidxv[b].at[j]],
                sems[b],
            ).wait()

    front_pk(0, 0)
    front_pk(1, 1)

    def zfill(i, _):
        zbuf[i, :] = jnp.zeros((2 * L,), jnp.bfloat16)
        return 0

    lax.fori_loop(0, ZROWS, zfill, 0)
    z0 = s * ZSTRIPE

    def zcopy(i, _):
        pltpu.async_copy(
            zbuf, shared.at[pl.ds(z0 + i * ZROWS, ZROWS)], ss0
        )
        return 0

    lax.fori_loop(0, ZSTRIPE // ZROWS, zcopy, 0)

    def zwait(i, _):
        pltpu.make_async_copy(
            zbuf, shared.at[pl.ds(z0, ZROWS)], ss0
        ).wait()
        return 0

    lax.fori_loop(0, ZSTRIPE // ZROWS, zwait, 0)
    front(0, 0)
    plsc.subcore_barrier()

    def round_body(m, _):
        for b in (0, 1):
            k = 2 * m + b
            cid = cid_of(k)

            @pl.when(cid < ANCHUNK)
            def _():
                wait_gathers(b)
                front(k + 1, 1 - b)

                @pl.when(k >= 2)
                def _():
                    wait_scatters(b)

                for t in range(CA // L):
                    j, tt = t // 8, t % 8
                    sidxv[b][j, pl.ds(tt * L, L)] = (
                        pkv[b][pl.ds(j * 384 + 128 + tt * L, L)]
                    )

                def fuse_body(eo, _):
                    for ei in range(4):
                        e = eo * 4 + ei
                        roff = (e >> 7) * 384 + 256 + (e & 127)
                        sr = roff + jnp.zeros((L,), jnp.int32)
                        rf = plsc.bitcast(
                            plsc.load_gather(pkv[b], [sr]), jnp.float32
                        )
                        rp = plsc.pack(
                            rf, rf, format=plsc.PackFormat.INTERLEAVED
                        )
                        brows[b][e, :] = rowsv[b][e, :] * rp
                    return 0

                lax.fori_loop(0, CA // 4, fuse_body, 0)

                for j in range(ANIDX):
                    pltpu.async_copy(
                        brows[b].at[pl.ds(j * 128, 128)],
                        shared.at[sidxv[b].at[j]],
                        sems[b],
                        add=True,
                    )
                front_pk(k + 2, b)
        return 0

    lax.fori_loop(0, ANITER // 2, round_body, 0)
    for b in (0, 1):
        wait_scatters(b)

    plsc.subcore_barrier()

    pltpu.sync_copy(
        shared.at[pl.ds(z0, ZSTRIPE)],
        part_out.at[pl.ds(c * HREGB + z0, ZSTRIPE)],
    )


@functools.partial(
    pl.pallas_call,
    out_shape=jax.ShapeDtypeStruct((NI, D), jnp.float32),
    grid=(NI // CBLK,),
    in_specs=[
        pl.BlockSpec((CBLK, D), lambda i: (i, 0)),
        pl.BlockSpec((CBLK, D), lambda i: (i + HREGB // CBLK, 0)),
        pl.BlockSpec((1, D), lambda i: (0, 0)),
    ],
    out_specs=pl.BlockSpec((CBLK, D), lambda i: (i, 0)),
)
def _combine(p0, p1, mu, h_out):
    x = p0[...].astype(jnp.float32) + p1[...].astype(jnp.float32) + mu[...]
    h_out[...] = 1.0 / (1.0 + jnp.exp(-x))


@functools.partial(
    pl.kernel,
    out_type=jax.ShapeDtypeStruct((E,), jnp.float32),
    mesh=_mesh,
    compiler_params=_params,
    scratch_types=[
        pltpu.VMEM((CB, D), jnp.float32),
        pltpu.VMEM((CB, D), jnp.float32),
        pltpu.VMEM((CB, WB), jnp.float32),
        pltpu.VMEM((CB, WB), jnp.float32),
        pltpu.VMEM((3 * CB,), jnp.int32),
        pltpu.VMEM((3 * CB,), jnp.int32),
        pltpu.VMEM((CB,), jnp.float32),
        pltpu.SemaphoreType.DMA,
        pltpu.SemaphoreType.DMA,
    ],
)
def _phase_b(pk3, h_hbm, wb_hbm, out1,
             h0, h1, w0, w1, pk0, pkb1, outv, sg0, sg1):
    c = lax.axis_index("c")
    s = lax.axis_index("s")
    wid = s * NC + c
    hrows = (h0, h1)
    wrows = (w0, w1)
    pkv = (pk0, pkb1)
    semg = (sg0, sg1)

    def cid_of(k):
        return wid + NW * k

    def front(k, b):
        cid = cid_of(k)

        @pl.when(cid < BNCHUNK)
        def _():
            bo = pl.multiple_of(cid * 3 * CB, 8)
            pltpu.sync_copy(pk3.at[pl.ds(bo, 3 * CB)], pkv[b])
            for j in range(BNIDX):
                pltpu.async_copy(
                    h_hbm.at[pkv[b].at[pl.ds(j * 384 + 128, 128)]],
                    hrows[b].at[pl.ds(j * 128, 128)],
                    semg[b],
                )
                pltpu.async_copy(
                    wb_hbm.at[pkv[b].at[pl.ds(j * 384, 128)]],
                    wrows[b].at[pl.ds(j * 128, 128)],
                    semg[b],
                )

    def wait_gathers(b):
        pltpu.make_async_copy(
            h_hbm.at[pl.ds(0, CB)], hrows[b], semg[b]
        ).wait()
        pltpu.make_async_copy(
            wb_hbm.at[pl.ds(0, CB)], wrows[b], semg[b]
        ).wait()

    front(0, 0)

    def round_body(m, _):
        for b in (0, 1):
            k = 2 * m + b
            cid = cid_of(k)

            @pl.when(cid < BNCHUNK)
            def _():
                wait_gathers(b)
                front(k + 1, 1 - b)

                def dot_body(g, _):
                    idx_e = g * L + _iota16()
                    fb = jnp.full((L,), D, jnp.int32)
                    acc = plsc.load_gather(wrows[b], [idx_e, fb])
                    for d in range(D):
                        fd = jnp.full((L,), d, jnp.int32)
                        hc = plsc.load_gather(hrows[b], [idx_e, fd])
                        wc = plsc.load_gather(wrows[b], [idx_e, fd])
                        acc = acc + hc * wc
                    outv[pl.ds(g * L, L)] = acc
                    return 0

                lax.fori_loop(0, CB // L, dot_body, 0)
                base = pl.multiple_of(cid * CB, 8)
                pltpu.sync_copy(outv, out1.at[pl.ds(base, CB)])
        return 0

    lax.fori_loop(0, BNITER // 2, round_body, 0)


def kernel(uid_in, iid_in, r_in, v, mu, w, b):
    uid1 = uid_in.astype(jnp.int32)
    iid1 = iid_in.astype(jnp.int32)
    pk3 = jnp.stack(
        [
            uid1.reshape(EG, 128),
            iid1.reshape(EG, 128),
            lax.bitcast_convert_type(r_in, jnp.int32).reshape(EG, 128),
        ],
        axis=1,
    ).reshape(3 * E)
    parts = _phase_a(pk3, v.astype(jnp.bfloat16))
    h = _combine(parts, parts, mu)
    wb = jnp.concatenate(
        [
            w,
            b[:, None],
            jnp.zeros((NU, WB - D - 1), jnp.float32),
        ],
        axis=1,
    )
    return _phase_b(pk3, h, wb)


# speedup vs baseline: 16.5912x; 1.1336x over previous
import functools

import jax
import jax.numpy as jnp
from jax import lax
from jax.experimental import pallas as pl
from jax.experimental.pallas import tpu as pltpu
from jax.experimental.pallas import tpu_sc as plsc

NU = 100000
NI = 100000
D = 32
E = 1600000

NC = 2
NS = 16
L = 16
NW = NC * NS
EG = E // 128

CA = 256
ANIDX = CA // 128
ANCHUNK = E // CA
ANITER = (ANCHUNK - 1) // NW + 1

CB = 512
BNIDX = CB // 128
BNCHUNK = E // CB
BNITER = (BNCHUNK - 1) // NW + 1

HREGB = 100000
ZSTRIPE = HREGB // NS
ZROWS = 50
CBLK = 4000
WB = 40

_mesh = plsc.VectorSubcoreMesh(
    core_axis_name="c", subcore_axis_name="s", num_cores=NC, num_subcores=NS
)
_params = pltpu.CompilerParams(
    needs_layout_passes=False, use_tc_tiling_on_sc=False
)


def _iota16():
    return lax.iota(jnp.int32, L)


@functools.partial(
    pl.kernel,
    out_type=jax.ShapeDtypeStruct((NC * HREGB, D), jnp.bfloat16),
    mesh=_mesh,
    compiler_params=_params,
    scratch_types=[
        pltpu.VMEM_SHARED((HREGB, D), jnp.bfloat16),
        pltpu.VMEM((CA, D), jnp.bfloat16),
        pltpu.VMEM((CA, D), jnp.bfloat16),
        pltpu.VMEM((CA, D), jnp.bfloat16),
        pltpu.VMEM((CA, D), jnp.bfloat16),
        pltpu.VMEM((3 * CA,), jnp.int32),
        pltpu.VMEM((3 * CA,), jnp.int32),
        pltpu.VMEM((ANIDX, 128), jnp.int32),
        pltpu.VMEM((ANIDX, 128), jnp.int32),
        pltpu.VMEM((ZROWS, D), jnp.bfloat16),
        pltpu.SemaphoreType.DMA,
        pltpu.SemaphoreType.DMA,
        pltpu.SemaphoreType.DMA,
        pltpu.SemaphoreType.DMA,
        pltpu.SemaphoreType.DMA,
        pltpu.SemaphoreType.DMA,
    ],
)
def _phase_a(pk3, v_hbm, part_out,
             shared, rows0, rows1, br0, br1, pk0, pkb1,
             si0, si1, zbuf, sg0, sg1, ss0, ss1, sp0, sp1):
    c = lax.axis_index("c")
    s = lax.axis_index("s")
    rowsv = (rows0, rows1)
    brows = (br0, br1)
    pkv = (pk0, pkb1)
    sidxv = (si0, si1)
    semg = (sg0, sg1)
    sems = (ss0, ss1)
    sempk = (sp0, sp1)

    def cid_of(k):
        return c + 2 * s + NW * k

    def front_pk(k, b):
        cid = cid_of(k)

        @pl.when(cid < ANCHUNK)
        def _():
            b768 = pl.multiple_of(cid * 3 * CA, 8)
            pltpu.async_copy(pk3.at[pl.ds(b768, 3 * CA)], pkv[b], sempk[b])

    def front(k, b):
        cid = cid_of(k)

        @pl.when(cid < ANCHUNK)
        def _():
            b768 = pl.multiple_of(cid * 3 * CA, 8)
            pltpu.make_async_copy(
                pk3.at[pl.ds(b768, 3 * CA)], pkv[b], sempk[b]
            ).wait()
            for j in range(ANIDX):
                pltpu.async_copy(
                    v_hbm.at[pkv[b].at[pl.ds(j * 384, 128)]],
                    rowsv[b].at[pl.ds(j * 128, 128)],
                    semg[b],
                )

    def wait_gathers(b):
        for j in range(ANIDX):
            pltpu.make_async_copy(
                v_hbm.at[pkv[b].at[pl.ds(j * 384, 128)]],
                rowsv[b].at[pl.ds(j * 128, 128)],
                semg[b],
            ).wait()

    def wait_scatters(b):
        for j in range(ANIDX):
            pltpu.make_async_copy(
                brows[b].at[pl.ds(j * 128, 128)],
                shared.at[sidxv[b].at[j]],
                sems[b],
            ).wait()

    front_pk(0, 0)
    front_pk(1, 1)

    def zfill(i, _):
        zbuf[i, :] = jnp.zeros((2 * L,), jnp.bfloat16)
        return 0

    lax.fori_loop(0, ZROWS, zfill, 0)
    z0 = s * ZSTRIPE

    def zcopy(i, _):
        pltpu.async_copy(
            zbuf, shared.at[pl.ds(z0 + i * ZROWS, ZROWS)], ss0
        )
        return 0

    lax.fori_loop(0, ZSTRIPE // ZROWS, zcopy, 0)

    def zwait(i, _):
        pltpu.make_async_copy(
            zbuf, shared.at[pl.ds(z0, ZROWS)], ss0
        ).wait()
        return 0

    lax.fori_loop(0, ZSTRIPE // ZROWS, zwait, 0)
    front(0, 0)
    plsc.subcore_barrier()

    def round_body(m, _):
        for b in (0, 1):
            k = 2 * m + b
            cid = cid_of(k)

            @pl.when(cid < ANCHUNK)
            def _():
                wait_gathers(b)
                front(k + 1, 1 - b)

                @pl.when(k >= 2)
                def _():
                    wait_scatters(b)

                for t in range(CA // L):
                    j, tt = t // 8, t % 8
                    sidxv[b][j, pl.ds(tt * L, L)] = (
                        pkv[b][pl.ds(j * 384 + 128 + tt * L, L)]
                    )

                def fuse_body(eo, _):
                    for ei in range(4):
                        e = eo * 4 + ei
                        roff = (e >> 7) * 384 + 256 + (e & 127)
                        sr = roff + jnp.zeros((L,), jnp.int32)
                        rf = plsc.bitcast(
                            plsc.load_gather(pkv[b], [sr]), jnp.float32
                        )
                        rp = plsc.pack(
                            rf, rf, format=plsc.PackFormat.INTERLEAVED
                        )
                        brows[b][e, :] = rowsv[b][e, :] * rp
                    return 0

                lax.fori_loop(0, CA // 4, fuse_body, 0)

                for j in range(ANIDX):
                    pltpu.async_copy(
                        brows[b].at[pl.ds(j * 128, 128)],
                        shared.at[sidxv[b].at[j]],
                        sems[b],
                        add=True,
                    )
                front_pk(k + 2, b)
        return 0

    lax.fori_loop(0, ANITER // 2, round_body, 0)
    for b in (0, 1):
        wait_scatters(b)

    plsc.subcore_barrier()

    pltpu.sync_copy(
        shared.at[pl.ds(z0, ZSTRIPE)],
        part_out.at[pl.ds(c * HREGB + z0, ZSTRIPE)],
    )


@functools.partial(
    pl.pallas_call,
    out_shape=jax.ShapeDtypeStruct((NI, D), jnp.float32),
    grid=(NI // CBLK,),
    in_specs=[
        pl.BlockSpec((CBLK, D), lambda i: (i, 0)),
        pl.BlockSpec((CBLK, D), lambda i: (i + HREGB // CBLK, 0)),
        pl.BlockSpec((1, D), lambda i: (0, 0)),
    ],
    out_specs=pl.BlockSpec((CBLK, D), lambda i: (i, 0)),
)
def _combine(p0, p1, mu, h_out):
    x = p0[...].astype(jnp.float32) + p1[...].astype(jnp.float32) + mu[...]
    h_out[...] = 1.0 / (1.0 + jnp.exp(-x))


@functools.partial(
    pl.kernel,
    out_type=jax.ShapeDtypeStruct((E,), jnp.float32),
    mesh=_mesh,
    compiler_params=_params,
    scratch_types=[
        pltpu.VMEM((CB, D), jnp.float32),
        pltpu.VMEM((CB, D), jnp.float32),
        pltpu.VMEM((CB, WB), jnp.float32),
        pltpu.VMEM((CB, WB), jnp.float32),
        pltpu.VMEM((3 * CB,), jnp.int32),
        pltpu.VMEM((3 * CB,), jnp.int32),
        pltpu.VMEM((CB,), jnp.float32),
        pltpu.SemaphoreType.DMA,
        pltpu.SemaphoreType.DMA,
    ],
)
def _phase_b(pk3, h_hbm, wb_hbm, out1,
             h0, h1, w0, w1, pk0, pkb1, outv, sg0, sg1):
    c = lax.axis_index("c")
    s = lax.axis_index("s")
    wid = s * NC + c
    hrows = (h0, h1)
    wrows = (w0, w1)
    pkv = (pk0, pkb1)
    semg = (sg0, sg1)

    def cid_of(k):
        return wid + NW * k

    def front(k, b):
        cid = cid_of(k)

        @pl.when(cid < BNCHUNK)
        def _():
            bo = pl.multiple_of(cid * 3 * CB, 8)
            pltpu.sync_copy(pk3.at[pl.ds(bo, 3 * CB)], pkv[b])
            for j in range(BNIDX):
                pltpu.async_copy(
                    h_hbm.at[pkv[b].at[pl.ds(j * 384 + 128, 128)]],
                    hrows[b].at[pl.ds(j * 128, 128)],
                    semg[b],
                )
                pltpu.async_copy(
                    wb_hbm.at[pkv[b].at[pl.ds(j * 384, 128)]],
                    wrows[b].at[pl.ds(j * 128, 128)],
                    semg[b],
                )

    def wait_gathers(b):
        pltpu.make_async_copy(
            h_hbm.at[pl.ds(0, CB)], hrows[b], semg[b]
        ).wait()
        pltpu.make_async_copy(
            wb_hbm.at[pl.ds(0, CB)], wrows[b], semg[b]
        ).wait()

    front(0, 0)

    def round_body(m, _):
        for b in (0, 1):
            k = 2 * m + b
            cid = cid_of(k)

            @pl.when(cid < BNCHUNK)
            def _():
                wait_gathers(b)
                front(k + 1, 1 - b)

                def dot_body(g, _):
                    idx_e = g * L + _iota16()
                    fb = jnp.full((L,), D, jnp.int32)
                    acc = plsc.load_gather(wrows[b], [idx_e, fb])
                    for d in range(D):
                        fd = jnp.full((L,), d, jnp.int32)
                        hc = plsc.load_gather(hrows[b], [idx_e, fd])
                        wc = plsc.load_gather(wrows[b], [idx_e, fd])
                        acc = acc + hc * wc
                    outv[pl.ds(g * L, L)] = acc
                    return 0

                lax.fori_loop(0, CB // L, dot_body, 0)
                base = pl.multiple_of(cid * CB, 8)
                pltpu.sync_copy(outv, out1.at[pl.ds(base, CB)])
        return 0

    lax.fori_loop(0, BNITER // 2, round_body, 0)


def kernel(uid_in, iid_in, r_in, v, mu, w, b):
    uid1 = uid_in.astype(jnp.int32)
    iid1 = iid_in.astype(jnp.int32)
    pk3 = jnp.stack(
        [
            uid1.reshape(EG, 128),
            iid1.reshape(EG, 128),
            lax.bitcast_convert_type(r_in, jnp.int32).reshape(EG, 128),
        ],
        axis=1,
    ).reshape(3 * E)
    parts = _phase_a(pk3, v.astype(jnp.bfloat16))
    h = _combine(parts, parts, mu)
    wb = jnp.concatenate(
        [
            w,
            b[:, None],
            jnp.zeros((NU, WB - D - 1), jnp.float32),
        ],
        axis=1,
    )
    return _phase_b(pk3, h, wb)
